# baseline (speedup 1.0000x reference)
import dataclasses
import functools

import jax
import jax.numpy as jnp
from jax import lax
from jax.experimental import pallas as pl
from jax.experimental.pallas import tpu as pltpu
from jax.experimental.pallas import tpu_sc as plsc

C = 128
NC, NS = 2, 16
NW = NC * NS
CHUNK = 128
LANES = 16
NBUF = 1


def _sc_mesh():
    return plsc.VectorSubcoreMesh(core_axis_name="c", subcore_axis_name="s")


def _sc_compiler_params():
    cp = pltpu.CompilerParams()
    if "needs_layout_passes" in pltpu.CompilerParams.__dataclass_fields__:
        cp = dataclasses.replace(cp, needs_layout_passes=False)
    return cp


def _padded_rows(n_rows):
    per_tile = ((n_rows + NS * CHUNK - 1) // (NS * CHUNK)) * CHUNK
    return per_tile, NS * per_tile


def _zero_2d(ref, nrows, width):
    @pl.loop(0, nrows)
    def _(i):
        @pl.loop(0, width, step=LANES)
        def _(j):
            ref[i, pl.ds(j, LANES)] = jnp.zeros((LANES,), jnp.float32)


def _make_sc_stats(n_rows, nnz):
    _, n_pad = _padded_rows(n_rows)
    hr = n_pad // CHUNK
    share = nnz // NS
    out_t = jax.ShapeDtypeStruct((hr, CHUNK), jnp.float32)

    @functools.partial(
        pl.kernel,
        out_type=(out_t, out_t, out_t),
        mesh=_sc_mesh(),
        compiler_params=_sc_compiler_params(),
        scratch_types=[
            pltpu.VMEM((share,), jnp.int32),
            pltpu.VMEM((share,), jnp.int32),
            pltpu.VMEM((hr, CHUNK), jnp.float32),
            pltpu.VMEM((hr, CHUNK), jnp.float32),
            pltpu.VMEM((hr, CHUNK), jnp.float32),
            pltpu.VMEM((hr,), jnp.int32),
            pltpu.VMEM_SHARED((hr, CHUNK), jnp.float32),
            pltpu.VMEM_SHARED((hr, CHUNK), jnp.float32),
            pltpu.VMEM_SHARED((hr, CHUNK), jnp.float32),
            pltpu.SemaphoreType.DMA,
            pltpu.SemaphoreType.DMA,
        ],
    )
    def stats(v_hbm, e_hbm, dv_hbm, cnt_hbm, de_hbm,
              vsh, esh, h1, h2, dvfull, ident, acc_dv, acc_cnt, acc_de,
              sem_v, sem_e):
        cid = lax.axis_index("c")
        sid = lax.axis_index("s")
        ones = jnp.ones((LANES,), jnp.float32)

        ld_v = pltpu.make_async_copy(
            v_hbm.at[pl.ds(sid * share, share)], vsh, sem_v)
        ld_e = pltpu.make_async_copy(
            e_hbm.at[pl.ds(sid * share, share)], esh, sem_e)
        ld_v.start()
        ld_e.start()

        _zero_2d(h1, hr, CHUNK)
        _zero_2d(h2, hr, CHUNK)

        @pl.loop(0, hr, step=LANES)
        def _(k):
            ident[pl.ds(k, LANES)] = lax.iota(jnp.int32, LANES) + k

        @pl.when(sid < hr // 8)
        def _():
            pltpu.sync_copy(h1.at[pl.ds(0, 8)], acc_dv.at[pl.ds(sid * 8, 8)])
            pltpu.sync_copy(h1.at[pl.ds(0, 8)], acc_cnt.at[pl.ds(sid * 8, 8)])
            pltpu.sync_copy(h1.at[pl.ds(0, 8)], acc_de.at[pl.ds(sid * 8, 8)])

        ld_v.wait()
        plsc.subcore_barrier()

        @pl.loop(0, share, step=LANES)
        def _(i):
            iv = vsh[pl.ds(i, LANES)]
            plsc.addupdate_scatter(
                h1, [jnp.right_shift(iv, 7), jnp.bitwise_and(iv, 127)], ones)

        pltpu.sync_copy(h1, acc_dv.at[ident], add=True)
        plsc.subcore_barrier()
        pltpu.sync_copy(acc_dv, dvfull)
        _zero_2d(h1, hr, CHUNK)
        ld_e.wait()

        @pl.loop(0, share, step=LANES)
        def _(i):
            iv = vsh[pl.ds(i, LANES)]
            ie = esh[pl.ds(i, LANES)]
            er, ec = jnp.right_shift(ie, 7), jnp.bitwise_and(ie, 127)
            plsc.addupdate_scatter(h1, [er, ec], ones)
            dvv = plsc.load_gather(
                dvfull, [jnp.right_shift(iv, 7), jnp.bitwise_and(iv, 127)])
            plsc.addupdate_scatter(h2, [er, ec], dvv)

        pltpu.sync_copy(h1, acc_cnt.at[ident], add=True)
        pltpu.sync_copy(h2, acc_de.at[ident], add=True)
        plsc.subcore_barrier()

        @pl.when(jnp.logical_and(cid == 0, sid < hr // 8))
        def _():
            pltpu.sync_copy(acc_dv.at[pl.ds(sid * 8, 8)],
                            dv_hbm.at[pl.ds(sid * 8, 8)])
            pltpu.sync_copy(acc_cnt.at[pl.ds(sid * 8, 8)],
                            cnt_hbm.at[pl.ds(sid * 8, 8)])
            pltpu.sync_copy(acc_de.at[pl.ds(sid * 8, 8)],
                            de_hbm.at[pl.ds(sid * 8, 8)])

    return stats


G = 128
SBC = 8


def _make_sc_segsum(n_rows, nsb):
    rpt, n_pad = _padded_rows(n_rows)

    @functools.partial(
        pl.kernel,
        out_type=jax.ShapeDtypeStruct((NC * n_pad, C), jnp.float32),
        mesh=_sc_mesh(),
        compiler_params=_sc_compiler_params(),
        scratch_types=(
            [pltpu.VMEM((SBC, G), jnp.int32) for _ in range(4)]
            + [pltpu.VMEM((G, C), jnp.float32) for _ in range(NBUF)]
            + [pltpu.VMEM_SHARED((n_pad, C), jnp.float32)]
            + [pltpu.SemaphoreType.DMA for _ in range(4 + 2 * NBUF)]
        ),
    )
    def segsum(table_hbm, src_hbm, dst_hbm, out_hbm, *refs):
        vmb = refs[0:2]
        emb = refs[2:4]
        rows = refs[4:4 + NBUF]
        acc_s = refs[4 + NBUF]
        isem = refs[5 + NBUF:9 + NBUF]
        gsem = refs[9 + NBUF:9 + 2 * NBUF]
        ssem = refs[9 + 2 * NBUF:9 + 3 * NBUF]
        cid = lax.axis_index("c")
        sid = lax.axis_index("s")
        wid = sid * NC + cid

        def idx_load(arr, buf, sem, s):
            return pltpu.make_async_copy(
                arr.at[pl.ds((wid * nsb + s) * SBC, SBC)], buf, sem)

        def gather(vb, k, b):
            return pltpu.make_async_copy(
                table_hbm.at[vb.at[k]], rows[b], gsem[b])

        def scatter(eb, k, b):
            return pltpu.make_async_copy(
                rows[b], acc_s.at[eb.at[k]], ssem[b])

        idx_load(src_hbm, vmb[0], isem[0], 0).start()
        idx_load(dst_hbm, emb[0], isem[1], 0).start()
        idx_load(src_hbm, vmb[1], isem[2], 1).start()
        idx_load(dst_hbm, emb[1], isem[3], 1).start()

        _zero_2d(rows[0], G, C)

        @pl.loop(0, rpt // G)
        def _(z):
            pltpu.sync_copy(rows[0], acc_s.at[pl.ds(sid * rpt + z * G, G)])

        plsc.subcore_barrier()

        @pl.loop(0, nsb // 2)
        def _(q):
            for p in range(2):
                s = 2 * q + p
                vb, eb = vmb[p], emb[p]
                idx_load(src_hbm, vb, isem[2 * p], s).wait()
                idx_load(dst_hbm, eb, isem[2 * p + 1], s).wait()

                for k in range(SBC):
                    gather(vb, k, 0).start()
                    gather(vb, k, 0).wait()
                    pltpu.sync_copy(rows[0], acc_s.at[eb.at[k]], add=True)

                @pl.when(s + 2 < nsb)
                def _():
                    idx_load(src_hbm, vb, isem[2 * p], s + 2).start()
                    idx_load(dst_hbm, eb, isem[2 * p + 1], s + 2).start()

        plsc.subcore_barrier()
        pltpu.sync_copy(
            acc_s.at[pl.ds(sid * rpt, rpt)],
            out_hbm.at[pl.ds(cid * n_pad + sid * rpt, rpt)],
        )

    return segsum


def _tc_linear(Xp, W_w, W_b):
    n = Xp.shape[0]

    def body(x_ref, w_ref, b_ref, out_ref):
        out_ref[...] = lax.dot_general(
            x_ref[...], w_ref[...], (((1,), (1,)), ((), ())),
            preferred_element_type=jnp.float32,
        ) + b_ref[...][None, :]

    return pl.pallas_call(
        body, out_shape=jax.ShapeDtypeStruct((n, C), jnp.float32)
    )(Xp, W_w, W_b)


def _tc_normalize(a0, a1, cnt, de_sum):
    n = a0.shape[0]

    def body(a0_ref, a1_ref, cnt_ref, de_ref, y_ref):
        sums = a0_ref[...] + a1_ref[...]
        cnt = cnt_ref[...]
        mean = jnp.where(cnt > 0, sums / jnp.maximum(cnt, 1.0), 0.0)
        de = de_ref[...] / (cnt + 1.0)
        de_r = jnp.where(cnt > 0, lax.rsqrt(jnp.maximum(de, 1e-30)), 1.0)
        y_ref[...] = de_r * mean

    return pl.pallas_call(
        body, out_shape=jax.ShapeDtypeStruct((n, C), jnp.float32)
    )(a0, a1, cnt, de_sum)


def _tc_finalize(b0, b1, dv):
    n = b0.shape[0]

    def body(b0_ref, b1_ref, dv_ref, out_ref):
        xn = b0_ref[...] + b1_ref[...]
        d = dv_ref[...]
        dv_r = jnp.where(d > 0, lax.rsqrt(jnp.maximum(d, 1.0)), 0.0)
        out_ref[...] = jnp.maximum(dv_r * xn, 0.0)

    return pl.pallas_call(
        body, out_shape=jax.ShapeDtypeStruct((n, C), jnp.float32)
    )(b0, b1, dv)


def kernel(X, hyperedge_index, S_features, W_w, W_b):
    del S_features
    n = X.shape[0]
    nnz = hyperedge_index.shape[1]
    V = hyperedge_index[0]
    E = hyperedge_index[1]
    _, n_pad = _padded_rows(n)

    grp = 2 * NW * SBC * G
    nnz_pad = ((nnz + grp - 1) // grp) * grp
    nsb = nnz_pad // (NW * SBC * G)
    sent = jnp.full((nnz_pad - nnz,), n_pad - 1, jnp.int32)
    V2 = jnp.concatenate([V, sent]).reshape(-1, G)
    E2 = jnp.concatenate([E, sent]).reshape(-1, G)

    dv, cnt, de = _make_sc_stats(n, nnz)(V, E)
    dv = dv.reshape(-1, 1)
    cnt = cnt.reshape(-1, 1)
    de = de.reshape(-1, 1)

    Xp = jnp.concatenate([X, jnp.zeros((n_pad - n, C), X.dtype)])
    xl = _tc_linear(Xp, W_w, W_b)

    acc_a = _make_sc_segsum(n, nsb)(xl, V2, E2)
    y = _tc_normalize(acc_a[:n_pad], acc_a[n_pad:], cnt, de)

    acc_b = _make_sc_segsum(n, nsb)(y, E2, V2)
    out = _tc_finalize(acc_b[:n_pad], acc_b[n_pad:], dv)
    return out[:n]


# speedup vs baseline: 1.1773x; 1.1773x over previous
import dataclasses
import functools

import jax
import jax.numpy as jnp
from jax import lax
from jax.experimental import pallas as pl
from jax.experimental.pallas import tpu as pltpu
from jax.experimental.pallas import tpu_sc as plsc

C = 128
NC, NS = 2, 16
NW = NC * NS
CHUNK = 128
LANES = 16
NBUF = 1


def _sc_mesh():
    return plsc.VectorSubcoreMesh(core_axis_name="c", subcore_axis_name="s")


def _sc_compiler_params():
    cp = pltpu.CompilerParams()
    if "needs_layout_passes" in pltpu.CompilerParams.__dataclass_fields__:
        cp = dataclasses.replace(cp, needs_layout_passes=False)
    return cp


def _padded_rows(n_rows):
    per_tile = ((n_rows + NS * CHUNK - 1) // (NS * CHUNK)) * CHUNK
    return per_tile, NS * per_tile


def _zero_2d(ref, nrows, width):
    @pl.loop(0, nrows)
    def _(i):
        @pl.loop(0, width, step=LANES)
        def _(j):
            ref[i, pl.ds(j, LANES)] = jnp.zeros((LANES,), jnp.float32)


def _make_sc_stats(n_rows, nnz):
    _, n_pad = _padded_rows(n_rows)
    hr = n_pad // CHUNK
    share = nnz // NS
    out_t = jax.ShapeDtypeStruct((hr, CHUNK), jnp.float32)

    @functools.partial(
        pl.kernel,
        out_type=(out_t, out_t, out_t),
        mesh=_sc_mesh(),
        compiler_params=_sc_compiler_params(),
        scratch_types=[
            pltpu.VMEM((share,), jnp.int32),
            pltpu.VMEM((share,), jnp.int32),
            pltpu.VMEM((hr, CHUNK), jnp.float32),
            pltpu.VMEM((hr, CHUNK), jnp.float32),
            pltpu.VMEM((hr, CHUNK), jnp.float32),
            pltpu.VMEM((hr,), jnp.int32),
            pltpu.VMEM_SHARED((hr, CHUNK), jnp.float32),
            pltpu.VMEM_SHARED((hr, CHUNK), jnp.float32),
            pltpu.VMEM_SHARED((hr, CHUNK), jnp.float32),
            pltpu.SemaphoreType.DMA,
            pltpu.SemaphoreType.DMA,
        ],
    )
    def stats(v_hbm, e_hbm, dv_hbm, cnt_hbm, de_hbm,
              vsh, esh, h1, h2, dvfull, ident, acc_dv, acc_cnt, acc_de,
              sem_v, sem_e):
        cid = lax.axis_index("c")
        sid = lax.axis_index("s")
        ones = jnp.ones((LANES,), jnp.float32)

        ld_v = pltpu.make_async_copy(
            v_hbm.at[pl.ds(sid * share, share)], vsh, sem_v)
        ld_e = pltpu.make_async_copy(
            e_hbm.at[pl.ds(sid * share, share)], esh, sem_e)
        ld_v.start()
        ld_e.start()

        _zero_2d(h1, hr, CHUNK)
        _zero_2d(h2, hr, CHUNK)

        @pl.loop(0, hr, step=LANES)
        def _(k):
            ident[pl.ds(k, LANES)] = lax.iota(jnp.int32, LANES) + k

        @pl.when(sid < hr // 8)
        def _():
            pltpu.sync_copy(h1.at[pl.ds(0, 8)], acc_dv.at[pl.ds(sid * 8, 8)])
            pltpu.sync_copy(h1.at[pl.ds(0, 8)], acc_cnt.at[pl.ds(sid * 8, 8)])
            pltpu.sync_copy(h1.at[pl.ds(0, 8)], acc_de.at[pl.ds(sid * 8, 8)])

        ld_v.wait()
        plsc.subcore_barrier()

        @pl.loop(0, share, step=LANES)
        def _(i):
            iv = vsh[pl.ds(i, LANES)]
            plsc.addupdate_scatter(
                h1, [jnp.right_shift(iv, 7), jnp.bitwise_and(iv, 127)], ones)

        pltpu.sync_copy(h1, acc_dv.at[ident], add=True)
        plsc.subcore_barrier()
        pltpu.sync_copy(acc_dv, dvfull)
        _zero_2d(h1, hr, CHUNK)
        ld_e.wait()

        @pl.loop(0, share, step=LANES)
        def _(i):
            iv = vsh[pl.ds(i, LANES)]
            ie = esh[pl.ds(i, LANES)]
            er, ec = jnp.right_shift(ie, 7), jnp.bitwise_and(ie, 127)
            plsc.addupdate_scatter(h1, [er, ec], ones)
            dvv = plsc.load_gather(
                dvfull, [jnp.right_shift(iv, 7), jnp.bitwise_and(iv, 127)])
            plsc.addupdate_scatter(h2, [er, ec], dvv)

        pltpu.sync_copy(h1, acc_cnt.at[ident], add=True)
        pltpu.sync_copy(h2, acc_de.at[ident], add=True)
        plsc.subcore_barrier()

        @pl.when(jnp.logical_and(cid == 0, sid < hr // 8))
        def _():
            pltpu.sync_copy(acc_dv.at[pl.ds(sid * 8, 8)],
                            dv_hbm.at[pl.ds(sid * 8, 8)])
            pltpu.sync_copy(acc_cnt.at[pl.ds(sid * 8, 8)],
                            cnt_hbm.at[pl.ds(sid * 8, 8)])
            pltpu.sync_copy(acc_de.at[pl.ds(sid * 8, 8)],
                            de_hbm.at[pl.ds(sid * 8, 8)])

    return stats


def _make_sc_segsum(n_rows, nch):
    rpt, n_pad = _padded_rows(n_rows)

    @functools.partial(
        pl.kernel,
        out_type=jax.ShapeDtypeStruct((NC * n_pad, C), jnp.float32),
        mesh=_sc_mesh(),
        compiler_params=_sc_compiler_params(),
        scratch_types=(
            [pltpu.VMEM((CHUNK,), jnp.int32) for _ in range(4)]
            + [pltpu.VMEM((CHUNK, C), jnp.float32)]
            + [pltpu.VMEM_SHARED((n_pad, C), jnp.float32)]
            + [pltpu.SemaphoreType.DMA for _ in range(5)]
        ),
    )
    def segsum(table_hbm, src_hbm, dst_hbm, out_hbm,
               sv0, sv1, dv0, dv1, rows, acc_s,
               iv0, iv1, ie0, ie1, gsem):
        cid = lax.axis_index("c")
        sid = lax.axis_index("s")
        wid = sid * NC + cid
        svb, dvb, ivs, ies = (sv0, sv1), (dv0, dv1), (iv0, iv1), (ie0, ie1)

        def ld_src(p, j):
            return pltpu.make_async_copy(
                src_hbm.at[pl.ds((wid + j * NW) * CHUNK, CHUNK)],
                svb[p], ivs[p])

        def ld_dst(p, j):
            return pltpu.make_async_copy(
                dst_hbm.at[pl.ds((wid + j * NW) * CHUNK, CHUNK)],
                dvb[p], ies[p])

        ld_src(0, 0).start()
        ld_dst(0, 0).start()
        ld_src(1, 1).start()
        ld_dst(1, 1).start()

        _zero_2d(rows, CHUNK, C)

        @pl.loop(0, rpt // CHUNK)
        def _(z):
            pltpu.sync_copy(
                rows, acc_s.at[pl.ds(sid * rpt + z * CHUNK, CHUNK)])

        plsc.subcore_barrier()

        @pl.loop(0, nch // 2)
        def _(q):
            for p in range(2):
                j = 2 * q + p
                ld_src(p, j).wait()
                ld_dst(p, j).wait()
                pltpu.async_copy(table_hbm.at[svb[p]], rows, gsem).wait()
                pltpu.sync_copy(rows, acc_s.at[dvb[p]], add=True)

                @pl.when(j + 2 < nch)
                def _():
                    ld_src(p, j + 2).start()
                    ld_dst(p, j + 2).start()

        plsc.subcore_barrier()
        pltpu.sync_copy(
            acc_s.at[pl.ds(sid * rpt, rpt)],
            out_hbm.at[pl.ds(cid * n_pad + sid * rpt, rpt)],
        )

    return segsum


def _tc_linear(Xp, W_w, W_b):
    n = Xp.shape[0]

    def body(x_ref, w_ref, b_ref, out_ref):
        out_ref[...] = lax.dot_general(
            x_ref[...], w_ref[...], (((1,), (1,)), ((), ())),
            preferred_element_type=jnp.float32,
        ) + b_ref[...][None, :]

    return pl.pallas_call(
        body, out_shape=jax.ShapeDtypeStruct((n, C), jnp.float32)
    )(Xp, W_w, W_b)


def _tc_normalize(a0, a1, cnt, de_sum):
    n = a0.shape[0]

    def body(a0_ref, a1_ref, cnt_ref, de_ref, y_ref):
        sums = a0_ref[...] + a1_ref[...]
        cnt = cnt_ref[...]
        mean = jnp.where(cnt > 0, sums / jnp.maximum(cnt, 1.0), 0.0)
        de = de_ref[...] / (cnt + 1.0)
        de_r = jnp.where(cnt > 0, lax.rsqrt(jnp.maximum(de, 1e-30)), 1.0)
        y_ref[...] = de_r * mean

    return pl.pallas_call(
        body, out_shape=jax.ShapeDtypeStruct((n, C), jnp.float32)
    )(a0, a1, cnt, de_sum)


def _tc_finalize(b0, b1, dv):
    n = b0.shape[0]

    def body(b0_ref, b1_ref, dv_ref, out_ref):
        xn = b0_ref[...] + b1_ref[...]
        d = dv_ref[...]
        dv_r = jnp.where(d > 0, lax.rsqrt(jnp.maximum(d, 1.0)), 0.0)
        out_ref[...] = jnp.maximum(dv_r * xn, 0.0)

    return pl.pallas_call(
        body, out_shape=jax.ShapeDtypeStruct((n, C), jnp.float32)
    )(b0, b1, dv)


def kernel(X, hyperedge_index, S_features, W_w, W_b):
    del S_features
    n = X.shape[0]
    nnz = hyperedge_index.shape[1]
    V = hyperedge_index[0]
    E = hyperedge_index[1]
    _, n_pad = _padded_rows(n)

    grp = 2 * NW * CHUNK
    nnz_pad = ((nnz + grp - 1) // grp) * grp
    nch = nnz_pad // (NW * CHUNK)
    sent = jnp.full((nnz_pad - nnz,), n_pad - 1, jnp.int32)
    V2 = jnp.concatenate([V, sent])
    E2 = jnp.concatenate([E, sent])

    dv, cnt, de = _make_sc_stats(n, nnz)(V, E)
    dv = dv.reshape(-1, 1)
    cnt = cnt.reshape(-1, 1)
    de = de.reshape(-1, 1)

    Xp = jnp.concatenate([X, jnp.zeros((n_pad - n, C), X.dtype)])
    xl = _tc_linear(Xp, W_w, W_b)

    acc_a = _make_sc_segsum(n, nch)(xl, V2, E2)
    y = _tc_normalize(acc_a[:n_pad], acc_a[n_pad:], cnt, de)

    acc_b = _make_sc_segsum(n, nch)(y, E2, V2)
    out = _tc_finalize(acc_b[:n_pad], acc_b[n_pad:], dv)
    return out[:n]


# speedup vs baseline: 2.6307x; 2.2345x over previous
import dataclasses
import functools

import jax
import jax.numpy as jnp
from jax import lax
from jax.experimental import pallas as pl
from jax.experimental.pallas import tpu as pltpu
from jax.experimental.pallas import tpu_sc as plsc

C = 128
NC, NS = 2, 16
NW = NC * NS
CHUNK = 128
LANES = 16
NBUF = 1


def _sc_mesh():
    return plsc.VectorSubcoreMesh(core_axis_name="c", subcore_axis_name="s")


def _sc_compiler_params():
    cp = pltpu.CompilerParams()
    if "needs_layout_passes" in pltpu.CompilerParams.__dataclass_fields__:
        cp = dataclasses.replace(cp, needs_layout_passes=False)
    return cp


def _padded_rows(n_rows):
    per_tile = ((n_rows + NS * CHUNK - 1) // (NS * CHUNK)) * CHUNK
    return per_tile, NS * per_tile


def _zero_2d(ref, nrows, width):
    @pl.loop(0, nrows)
    def _(i):
        @pl.loop(0, width, step=LANES)
        def _(j):
            ref[i, pl.ds(j, LANES)] = jnp.zeros((LANES,), jnp.float32)


def _make_sc_stats(n_rows, nnz):
    _, n_pad = _padded_rows(n_rows)
    hr = n_pad // CHUNK
    share = nnz // NS
    out_t = jax.ShapeDtypeStruct((hr, CHUNK), jnp.float32)

    @functools.partial(
        pl.kernel,
        out_type=(out_t, out_t, out_t),
        mesh=_sc_mesh(),
        compiler_params=_sc_compiler_params(),
        scratch_types=[
            pltpu.VMEM((share,), jnp.int32),
            pltpu.VMEM((share,), jnp.int32),
            pltpu.VMEM((hr, CHUNK), jnp.float32),
            pltpu.VMEM((hr, CHUNK), jnp.float32),
            pltpu.VMEM((hr, CHUNK), jnp.float32),
            pltpu.VMEM((hr,), jnp.int32),
            pltpu.VMEM_SHARED((hr, CHUNK), jnp.float32),
            pltpu.VMEM_SHARED((hr, CHUNK), jnp.float32),
            pltpu.VMEM_SHARED((hr, CHUNK), jnp.float32),
            pltpu.SemaphoreType.DMA,
            pltpu.SemaphoreType.DMA,
        ],
    )
    def stats(v_hbm, e_hbm, dv_hbm, cnt_hbm, de_hbm,
              vsh, esh, h1, h2, dvfull, ident, acc_dv, acc_cnt, acc_de,
              sem_v, sem_e):
        cid = lax.axis_index("c")
        sid = lax.axis_index("s")
        ones = jnp.ones((LANES,), jnp.float32)

        ld_v = pltpu.make_async_copy(
            v_hbm.at[pl.ds(sid * share, share)], vsh, sem_v)
        ld_e = pltpu.make_async_copy(
            e_hbm.at[pl.ds(sid * share, share)], esh, sem_e)
        ld_v.start()
        ld_e.start()

        _zero_2d(h1, hr, CHUNK)
        _zero_2d(h2, hr, CHUNK)

        @pl.loop(0, hr, step=LANES)
        def _(k):
            ident[pl.ds(k, LANES)] = lax.iota(jnp.int32, LANES) + k

        @pl.when(sid < hr // 8)
        def _():
            pltpu.sync_copy(h1.at[pl.ds(0, 8)], acc_dv.at[pl.ds(sid * 8, 8)])
            pltpu.sync_copy(h1.at[pl.ds(0, 8)], acc_cnt.at[pl.ds(sid * 8, 8)])
            pltpu.sync_copy(h1.at[pl.ds(0, 8)], acc_de.at[pl.ds(sid * 8, 8)])

        ld_v.wait()
        plsc.subcore_barrier()

        @pl.loop(0, share, step=LANES)
        def _(i):
            iv = vsh[pl.ds(i, LANES)]
            plsc.addupdate_scatter(
                h1, [jnp.right_shift(iv, 7), jnp.bitwise_and(iv, 127)], ones)

        pltpu.sync_copy(h1, acc_dv.at[ident], add=True)
        plsc.subcore_barrier()
        pltpu.sync_copy(acc_dv, dvfull)
        _zero_2d(h1, hr, CHUNK)
        ld_e.wait()

        @pl.loop(0, share, step=LANES)
        def _(i):
            iv = vsh[pl.ds(i, LANES)]
            ie = esh[pl.ds(i, LANES)]
            er, ec = jnp.right_shift(ie, 7), jnp.bitwise_and(ie, 127)
            plsc.addupdate_scatter(h1, [er, ec], ones)
            dvv = plsc.load_gather(
                dvfull, [jnp.right_shift(iv, 7), jnp.bitwise_and(iv, 127)])
            plsc.addupdate_scatter(h2, [er, ec], dvv)

        pltpu.sync_copy(h1, acc_cnt.at[ident], add=True)
        pltpu.sync_copy(h2, acc_de.at[ident], add=True)
        plsc.subcore_barrier()

        @pl.when(jnp.logical_and(cid == 0, sid < hr // 8))
        def _():
            pltpu.sync_copy(acc_dv.at[pl.ds(sid * 8, 8)],
                            dv_hbm.at[pl.ds(sid * 8, 8)])
            pltpu.sync_copy(acc_cnt.at[pl.ds(sid * 8, 8)],
                            cnt_hbm.at[pl.ds(sid * 8, 8)])
            pltpu.sync_copy(acc_de.at[pl.ds(sid * 8, 8)],
                            de_hbm.at[pl.ds(sid * 8, 8)])

    return stats


def _make_sc_segsum(n_rows, nch):
    rpt, n_pad = _padded_rows(n_rows)

    @functools.partial(
        pl.kernel,
        out_type=jax.ShapeDtypeStruct((NC * n_pad, C), jnp.float32),
        mesh=_sc_mesh(),
        compiler_params=_sc_compiler_params(),
        scratch_types=(
            [pltpu.VMEM((CHUNK,), jnp.int32) for _ in range(4)]
            + [pltpu.VMEM((CHUNK, C), jnp.float32)]
            + [pltpu.VMEM_SHARED((n_pad, C), jnp.float32)]
            + [pltpu.SemaphoreType.DMA for _ in range(5)]
        ),
    )
    def segsum(table_hbm, src_hbm, dst_hbm, out_hbm,
               sv0, sv1, dv0, dv1, rows, acc_s,
               iv0, iv1, ie0, ie1, gsem):
        cid = lax.axis_index("c")
        sid = lax.axis_index("s")
        wid = sid * NC + cid
        svb, dvb, ivs, ies = (sv0, sv1), (dv0, dv1), (iv0, iv1), (ie0, ie1)

        def ld_src(p, j):
            return pltpu.make_async_copy(
                src_hbm.at[pl.ds((wid + j * NW) * CHUNK, CHUNK)],
                svb[p], ivs[p])

        def ld_dst(p, j):
            return pltpu.make_async_copy(
                dst_hbm.at[pl.ds((wid + j * NW) * CHUNK, CHUNK)],
                dvb[p], ies[p])

        ld_src(0, 0).start()
        ld_dst(0, 0).start()
        ld_src(1, 1).start()
        ld_dst(1, 1).start()

        _zero_2d(rows, CHUNK, C)

        @pl.loop(0, rpt // CHUNK)
        def _(z):
            pltpu.sync_copy(
                rows, acc_s.at[pl.ds(sid * rpt + z * CHUNK, CHUNK)])

        plsc.subcore_barrier()

        @pl.loop(0, nch // 2)
        def _(q):
            for p in range(2):
                j = 2 * q + p
                ld_src(p, j).wait()
                ld_dst(p, j).wait()
                pltpu.async_copy(table_hbm.at[svb[p]], rows, gsem).wait()
                pltpu.sync_copy(rows, acc_s.at[dvb[p]], add=True)

                @pl.when(j + 2 < nch)
                def _():
                    ld_src(p, j + 2).start()
                    ld_dst(p, j + 2).start()

        plsc.subcore_barrier()
        pltpu.sync_copy(
            acc_s.at[pl.ds(sid * rpt, rpt)],
            out_hbm.at[pl.ds(cid * n_pad + sid * rpt, rpt)],
        )

    return segsum


def _tc_linear(Xp, W_w, W_b):
    n = Xp.shape[0]

    def body(x_ref, w_ref, b_ref, out_ref):
        out_ref[...] = lax.dot_general(
            x_ref[...], w_ref[...], (((1,), (1,)), ((), ())),
            preferred_element_type=jnp.float32,
        ) + b_ref[...][None, :]

    return pl.pallas_call(
        body, out_shape=jax.ShapeDtypeStruct((n, C), jnp.float32)
    )(Xp, W_w, W_b)


def _tc_normalize(a0, a1, cnt, de_sum):
    n = a0.shape[0]

    def body(a0_ref, a1_ref, cnt_ref, de_ref, y_ref):
        sums = a0_ref[...] + a1_ref[...]
        cnt = cnt_ref[...]
        mean = jnp.where(cnt > 0, sums / jnp.maximum(cnt, 1.0), 0.0)
        de = de_ref[...] / (cnt + 1.0)
        de_r = jnp.where(cnt > 0, lax.rsqrt(jnp.maximum(de, 1e-30)), 1.0)
        y_ref[...] = de_r * mean

    return pl.pallas_call(
        body, out_shape=jax.ShapeDtypeStruct((n, C), jnp.float32)
    )(a0, a1, cnt, de_sum)


def _tc_finalize(b0, b1, dv):
    n = b0.shape[0]

    def body(b0_ref, b1_ref, dv_ref, out_ref):
        xn = b0_ref[...] + b1_ref[...]
        d = dv_ref[...]
        dv_r = jnp.where(d > 0, lax.rsqrt(jnp.maximum(d, 1.0)), 0.0)
        out_ref[...] = jnp.maximum(dv_r * xn, 0.0)

    return pl.pallas_call(
        body, out_shape=jax.ShapeDtypeStruct((n, C), jnp.float32)
    )(b0, b1, dv)


def kernel(X, hyperedge_index, S_features, W_w, W_b):
    del S_features
    n = X.shape[0]
    nnz = hyperedge_index.shape[1]
    V = hyperedge_index[0]
    E = hyperedge_index[1]
    _, n_pad = _padded_rows(n)

    grp = 2 * NW * CHUNK
    nnz_pad = ((nnz + grp - 1) // grp) * grp
    nch = nnz_pad // (NW * CHUNK)
    sent = n + jnp.arange(nnz_pad - nnz, dtype=jnp.int32) % (n_pad - n)
    V2 = jnp.concatenate([V, sent])
    E2 = jnp.concatenate([E, sent])

    dv, cnt, de = _make_sc_stats(n, nnz)(V, E)
    dv = dv.reshape(-1, 1)
    cnt = cnt.reshape(-1, 1)
    de = de.reshape(-1, 1)

    Xp = jnp.concatenate([X, jnp.zeros((n_pad - n, C), X.dtype)])
    xl = _tc_linear(Xp, W_w, W_b)

    acc_a = _make_sc_segsum(n, nch)(xl, V2, E2)
    y = _tc_normalize(acc_a[:n_pad], acc_a[n_pad:], cnt, de)

    acc_b = _make_sc_segsum(n, nch)(y, E2, V2)
    out = _tc_finalize(acc_b[:n_pad], acc_b[n_pad:], dv)
    return out[:n]


# speedup vs baseline: 3.3247x; 1.2638x over previous
import dataclasses
import functools

import jax
import jax.numpy as jnp
from jax import lax
from jax.experimental import pallas as pl
from jax.experimental.pallas import tpu as pltpu
from jax.experimental.pallas import tpu_sc as plsc

C = 128
NC, NS = 2, 16
NW = NC * NS
CHUNK = 128
LANES = 16
NBUF = 1


def _sc_mesh():
    return plsc.VectorSubcoreMesh(core_axis_name="c", subcore_axis_name="s")


def _sc_compiler_params():
    cp = pltpu.CompilerParams()
    if "needs_layout_passes" in pltpu.CompilerParams.__dataclass_fields__:
        cp = dataclasses.replace(cp, needs_layout_passes=False)
    return cp


def _padded_rows(n_rows):
    per_tile = ((n_rows + NS * CHUNK - 1) // (NS * CHUNK)) * CHUNK
    return per_tile, NS * per_tile


def _zero_2d(ref, nrows, width):
    @pl.loop(0, nrows)
    def _(i):
        @pl.loop(0, width, step=LANES)
        def _(j):
            ref[i, pl.ds(j, LANES)] = jnp.zeros((LANES,), jnp.float32)


def _make_sc_stats(n_rows, nnz):
    _, n_pad = _padded_rows(n_rows)
    hr = n_pad // CHUNK
    share = nnz // NS
    out_t = jax.ShapeDtypeStruct((hr, CHUNK), jnp.float32)

    @functools.partial(
        pl.kernel,
        out_type=(out_t, out_t, out_t),
        mesh=_sc_mesh(),
        compiler_params=_sc_compiler_params(),
        scratch_types=[
            pltpu.VMEM((share,), jnp.int32),
            pltpu.VMEM((share,), jnp.int32),
            pltpu.VMEM((hr, CHUNK), jnp.float32),
            pltpu.VMEM((hr, CHUNK), jnp.float32),
            pltpu.VMEM((hr, CHUNK), jnp.float32),
            pltpu.VMEM((hr,), jnp.int32),
            pltpu.VMEM_SHARED((hr, CHUNK), jnp.float32),
            pltpu.VMEM_SHARED((hr, CHUNK), jnp.float32),
            pltpu.VMEM_SHARED((hr, CHUNK), jnp.float32),
            pltpu.SemaphoreType.DMA,
            pltpu.SemaphoreType.DMA,
        ],
    )
    def stats(v_hbm, e_hbm, dv_hbm, cnt_hbm, de_hbm,
              vsh, esh, h1, h2, dvfull, ident, acc_dv, acc_cnt, acc_de,
              sem_v, sem_e):
        cid = lax.axis_index("c")
        sid = lax.axis_index("s")
        ones = jnp.ones((LANES,), jnp.float32)

        ld_v = pltpu.make_async_copy(
            v_hbm.at[pl.ds(sid * share, share)], vsh, sem_v)
        ld_e = pltpu.make_async_copy(
            e_hbm.at[pl.ds(sid * share, share)], esh, sem_e)
        ld_v.start()
        ld_e.start()

        _zero_2d(h1, hr, CHUNK)
        _zero_2d(h2, hr, CHUNK)

        @pl.loop(0, hr, step=LANES)
        def _(k):
            ident[pl.ds(k, LANES)] = lax.iota(jnp.int32, LANES) + k

        @pl.when(sid < hr // 8)
        def _():
            pltpu.sync_copy(h1.at[pl.ds(0, 8)], acc_dv.at[pl.ds(sid * 8, 8)])
            pltpu.sync_copy(h1.at[pl.ds(0, 8)], acc_cnt.at[pl.ds(sid * 8, 8)])
            pltpu.sync_copy(h1.at[pl.ds(0, 8)], acc_de.at[pl.ds(sid * 8, 8)])

        ld_v.wait()
        plsc.subcore_barrier()

        @pl.loop(0, share, step=LANES)
        def _(i):
            iv = vsh[pl.ds(i, LANES)]
            plsc.addupdate_scatter(
                h1, [jnp.right_shift(iv, 7), jnp.bitwise_and(iv, 127)], ones)

        pltpu.sync_copy(h1, acc_dv.at[ident], add=True)
        plsc.subcore_barrier()
        pltpu.sync_copy(acc_dv, dvfull)
        _zero_2d(h1, hr, CHUNK)
        ld_e.wait()

        @pl.loop(0, share, step=LANES)
        def _(i):
            iv = vsh[pl.ds(i, LANES)]
            ie = esh[pl.ds(i, LANES)]
            er, ec = jnp.right_shift(ie, 7), jnp.bitwise_and(ie, 127)
            plsc.addupdate_scatter(h1, [er, ec], ones)
            dvv = plsc.load_gather(
                dvfull, [jnp.right_shift(iv, 7), jnp.bitwise_and(iv, 127)])
            plsc.addupdate_scatter(h2, [er, ec], dvv)

        pltpu.sync_copy(h1, acc_cnt.at[ident], add=True)
        pltpu.sync_copy(h2, acc_de.at[ident], add=True)
        plsc.subcore_barrier()

        @pl.when(jnp.logical_and(cid == 0, sid < hr // 8))
        def _():
            pltpu.sync_copy(acc_dv.at[pl.ds(sid * 8, 8)],
                            dv_hbm.at[pl.ds(sid * 8, 8)])
            pltpu.sync_copy(acc_cnt.at[pl.ds(sid * 8, 8)],
                            cnt_hbm.at[pl.ds(sid * 8, 8)])
            pltpu.sync_copy(acc_de.at[pl.ds(sid * 8, 8)],
                            de_hbm.at[pl.ds(sid * 8, 8)])

    return stats


def _make_sc_segsum(n_rows, nch):
    rpt, n_pad = _padded_rows(n_rows)

    @functools.partial(
        pl.kernel,
        out_type=jax.ShapeDtypeStruct((NC * n_pad, C), jnp.float32),
        mesh=_sc_mesh(),
        compiler_params=_sc_compiler_params(),
        scratch_types=(
            [pltpu.VMEM((CHUNK,), jnp.int32) for _ in range(8)]
            + [pltpu.VMEM((CHUNK, C), jnp.float32) for _ in range(2)]
            + [pltpu.VMEM_SHARED((n_pad, C), jnp.float32)]
            + [pltpu.SemaphoreType.DMA for _ in range(10)]
        ),
    )
    def segsum(table_hbm, src_hbm, dst_hbm, out_hbm, *refs):
        svb, dvb = refs[0:4], refs[4:8]
        rows = refs[8:10]
        acc_s = refs[10]
        ivs, ies = refs[11:15], refs[15:19]
        gsem = refs[19:21]
        cid = lax.axis_index("c")
        sid = lax.axis_index("s")
        wid = sid * NC + cid

        def ld_src(p, j):
            return pltpu.make_async_copy(
                src_hbm.at[pl.ds((wid + j * NW) * CHUNK, CHUNK)],
                svb[p], ivs[p])

        def ld_dst(p, j):
            return pltpu.make_async_copy(
                dst_hbm.at[pl.ds((wid + j * NW) * CHUNK, CHUNK)],
                dvb[p], ies[p])

        def gather(p, b):
            return pltpu.make_async_copy(
                table_hbm.at[svb[p]], rows[b], gsem[b])

        for p in range(3):
            ld_src(p, p).start()
            ld_dst(p, p).start()

        _zero_2d(rows[0], CHUNK, C)

        @pl.loop(0, rpt // CHUNK)
        def _(z):
            pltpu.sync_copy(
                rows[0], acc_s.at[pl.ds(sid * rpt + z * CHUNK, CHUNK)])

        plsc.subcore_barrier()

        ld_src(0, 0).wait()
        gather(0, 0).start()

        @pl.loop(0, nch // 4)
        def _(q):
            for p in range(4):
                j = 4 * q + p
                gather(p, p % 2).wait()

                @pl.when(j + 1 < nch)
                def _():
                    ld_src((p + 1) % 4, 0).wait()
                    gather((p + 1) % 4, (p + 1) % 2).start()

                ld_dst(p, 0).wait()
                pltpu.sync_copy(rows[p % 2], acc_s.at[dvb[p]], add=True)

                @pl.when(j + 3 < nch)
                def _():
                    ld_src((p + 3) % 4, j + 3).start()
                    ld_dst((p + 3) % 4, j + 3).start()

        plsc.subcore_barrier()
        pltpu.sync_copy(
            acc_s.at[pl.ds(sid * rpt, rpt)],
            out_hbm.at[pl.ds(cid * n_pad + sid * rpt, rpt)],
        )

    return segsum


def _tc_linear(Xp, W_w, W_b):
    n = Xp.shape[0]

    def body(x_ref, w_ref, b_ref, out_ref):
        out_ref[...] = lax.dot_general(
            x_ref[...], w_ref[...], (((1,), (1,)), ((), ())),
            preferred_element_type=jnp.float32,
        ) + b_ref[...][None, :]

    return pl.pallas_call(
        body, out_shape=jax.ShapeDtypeStruct((n, C), jnp.float32)
    )(Xp, W_w, W_b)


def _tc_normalize(a0, a1, cnt, de_sum):
    n = a0.shape[0]

    def body(a0_ref, a1_ref, cnt_ref, de_ref, y_ref):
        sums = a0_ref[...] + a1_ref[...]
        cnt = cnt_ref[...]
        mean = jnp.where(cnt > 0, sums / jnp.maximum(cnt, 1.0), 0.0)
        de = de_ref[...] / (cnt + 1.0)
        de_r = jnp.where(cnt > 0, lax.rsqrt(jnp.maximum(de, 1e-30)), 1.0)
        y_ref[...] = de_r * mean

    return pl.pallas_call(
        body, out_shape=jax.ShapeDtypeStruct((n, C), jnp.float32)
    )(a0, a1, cnt, de_sum)


def _tc_finalize(b0, b1, dv):
    n = b0.shape[0]

    def body(b0_ref, b1_ref, dv_ref, out_ref):
        xn = b0_ref[...] + b1_ref[...]
        d = dv_ref[...]
        dv_r = jnp.where(d > 0, lax.rsqrt(jnp.maximum(d, 1.0)), 0.0)
        out_ref[...] = jnp.maximum(dv_r * xn, 0.0)

    return pl.pallas_call(
        body, out_shape=jax.ShapeDtypeStruct((n, C), jnp.float32)
    )(b0, b1, dv)


def kernel(X, hyperedge_index, S_features, W_w, W_b):
    del S_features
    n = X.shape[0]
    nnz = hyperedge_index.shape[1]
    V = hyperedge_index[0]
    E = hyperedge_index[1]
    _, n_pad = _padded_rows(n)

    grp = 4 * NW * CHUNK
    nnz_pad = ((nnz + grp - 1) // grp) * grp
    nch = nnz_pad // (NW * CHUNK)
    sent = n + jnp.arange(nnz_pad - nnz, dtype=jnp.int32) % (n_pad - n)
    V2 = jnp.concatenate([V, sent])
    E2 = jnp.concatenate([E, sent])

    dv, cnt, de = _make_sc_stats(n, nnz)(V, E)
    dv = dv.reshape(-1, 1)
    cnt = cnt.reshape(-1, 1)
    de = de.reshape(-1, 1)

    Xp = jnp.concatenate([X, jnp.zeros((n_pad - n, C), X.dtype)])
    xl = _tc_linear(Xp, W_w, W_b)

    acc_a = _make_sc_segsum(n, nch)(xl, V2, E2)
    y = _tc_normalize(acc_a[:n_pad], acc_a[n_pad:], cnt, de)

    acc_b = _make_sc_segsum(n, nch)(y, E2, V2)
    out = _tc_finalize(acc_b[:n_pad], acc_b[n_pad:], dv)
    return out[:n]
